# baseline (speedup 1.0000x reference)
import functools

import jax
import jax.numpy as jnp
from jax import lax
from jax.experimental import pallas as pl
from jax.experimental.pallas import tpu as pltpu
from jax.experimental.pallas import tpu_sc as plsc

NUM_CLASSES = 1000
BATCH = 16384
NC = 2
NS = 16
NW = NC * NS
RPW = BATCH // NW
CH = 64
NCH = RPW // CH
CHW = CH * NUM_CLASSES


def _sc_body(x_hbm, out_hbm, xv, buf, grp):
    wid = lax.axis_index("s") * NC + lax.axis_index("c")
    base = wid * RPW
    pltpu.sync_copy(x_hbm.at[pl.ds(pl.multiple_of(base, RPW), RPW)], xv)

    zeros16 = jnp.zeros((16,), jnp.float32)
    ones16 = jnp.ones((16,), jnp.float32)
    lane = lax.iota(jnp.int32, 16)
    row1000 = lane * NUM_CLASSES

    @plsc.parallel_loop(0, CHW, step=16, unroll=8)
    def _zero(i):
        buf[pl.ds(pl.multiple_of(i, 16), 16)] = zeros16

    for k in range(NCH):
        for j in range(CH // 16):
            xs = xv[pl.ds(k * CH + j * 16, 16)]
            idx = row1000 + j * 16 * NUM_CLASSES + xs
            plsc.store_scatter(buf, [idx], ones16)
        pltpu.sync_copy(
            buf, out_hbm.at[pl.ds(pl.multiple_of((base + k * CH) * NUM_CLASSES, 8), CHW)]
        )
        for j in range(CH // 16):
            xs = xv[pl.ds(k * CH + j * 16, 16)]
            idx = row1000 + j * 16 * NUM_CLASSES + xs
            plsc.store_scatter(buf, [idx], zeros16)


def kernel(x):
    x = x.astype(jnp.int32)
    mesh = plsc.VectorSubcoreMesh(core_axis_name="c", subcore_axis_name="s")
    flat = pl.kernel(
        _sc_body,
        out_type=jax.ShapeDtypeStruct((BATCH * NUM_CLASSES,), jnp.float32),
        mesh=mesh,
        scratch_types=[
            pltpu.VMEM((RPW,), jnp.int32),
            pltpu.VMEM((CHW,), jnp.float32),
            pltpu.VMEM((16,), jnp.int32),
        ],
        compiler_params=pltpu.CompilerParams(needs_layout_passes=False),
    )(x)
    return flat.reshape(BATCH, NUM_CLASSES)


# speedup vs baseline: 3.3173x; 3.3173x over previous
import jax
import jax.numpy as jnp
from jax import lax
from jax.experimental import pallas as pl
from jax.experimental.pallas import tpu as pltpu
from jax.experimental.pallas import tpu_sc as plsc

NUM_CLASSES = 1000
BATCH = 16384
NC = 2
NS = 16
NW = NC * NS
CPW = BATCH // NW
CH = 200
NCH = NUM_CLASSES // CH


def _sc_body(x_hbm, out_hbm, xv, buf):
    wid = lax.axis_index("s") * NC + lax.axis_index("c")
    col0 = wid * CPW
    pltpu.sync_copy(x_hbm.at[pl.ds(pl.multiple_of(col0, CPW), CPW)], xv)

    zeros16 = jnp.zeros((16,), jnp.float32)
    ones16 = jnp.ones((16,), jnp.float32)
    lane = lax.iota(jnp.int32, 16)

    @plsc.parallel_loop(0, CH, step=1, unroll=2)
    def _zero(r):
        for c in range(CPW // 16):
            buf[r, pl.ds(pl.multiple_of(c * 16, 16), 16)] = zeros16

    for k in range(NCH):
        for j in range(CPW // 16):
            xs = xv[pl.ds(j * 16, 16)]
            rloc = xs - k * CH
            mask = (rloc >= 0) & (rloc < CH)
            plsc.store_scatter(buf, [rloc, lane + j * 16], ones16, mask=mask)
        pltpu.sync_copy(
            buf,
            out_hbm.at[pl.ds(k * CH, CH), pl.ds(pl.multiple_of(col0, CPW), CPW)],
        )
        if k < NCH - 1:
            for j in range(CPW // 16):
                xs = xv[pl.ds(j * 16, 16)]
                rloc = xs - k * CH
                mask = (rloc >= 0) & (rloc < CH)
                plsc.store_scatter(buf, [rloc, lane + j * 16], zeros16, mask=mask)


def kernel(x):
    x = x.astype(jnp.int32)
    mesh = plsc.VectorSubcoreMesh(core_axis_name="c", subcore_axis_name="s")
    oh_t = pl.kernel(
        _sc_body,
        out_type=jax.ShapeDtypeStruct((NUM_CLASSES, BATCH), jnp.float32),
        mesh=mesh,
        scratch_types=[
            pltpu.VMEM((CPW,), jnp.int32),
            pltpu.VMEM((CH, CPW), jnp.float32),
        ],
        compiler_params=pltpu.CompilerParams(needs_layout_passes=False),
    )(x)
    return oh_t.T


# speedup vs baseline: 7.2310x; 2.1798x over previous
import jax
import jax.numpy as jnp
from jax import lax
from jax.experimental import pallas as pl
from jax.experimental.pallas import tpu as pltpu

NUM_CLASSES = 1000
BATCH = 16384
COLS = 1024
NBLK = BATCH // COLS
NBUF = 4


def _onehot_manual(x_ref, out_ref, xv_ref, scratch_ref, xsem, sems):
    pltpu.make_async_copy(x_ref, xv_ref, xsem).start()
    pltpu.make_async_copy(x_ref, xv_ref, xsem).wait()
    rows = lax.broadcasted_iota(jnp.int32, (NUM_CLASSES, COLS), 0)
    for i in range(NBLK):
        buf = i % NBUF
        if i >= NBUF:
            pltpu.make_async_copy(
                scratch_ref.at[buf],
                out_ref.at[:, pl.ds((i - NBUF) * COLS, COLS)],
                sems.at[buf],
            ).wait()
        x = xv_ref[:, pl.ds(i * COLS, COLS)]
        scratch_ref[buf] = jnp.where(x == rows, 1.0, 0.0).astype(jnp.float32)
        pltpu.make_async_copy(
            scratch_ref.at[buf],
            out_ref.at[:, pl.ds(i * COLS, COLS)],
            sems.at[buf],
        ).start()
    for i in range(NBLK - NBUF, NBLK):
        buf = i % NBUF
        pltpu.make_async_copy(
            scratch_ref.at[buf],
            out_ref.at[:, pl.ds(i * COLS, COLS)],
            sems.at[buf],
        ).wait()


def kernel(x):
    x = x.astype(jnp.int32).reshape(1, BATCH)
    oh_t = pl.pallas_call(
        _onehot_manual,
        in_specs=[pl.BlockSpec(memory_space=pl.ANY)],
        out_specs=pl.BlockSpec(memory_space=pl.ANY),
        out_shape=jax.ShapeDtypeStruct((NUM_CLASSES, BATCH), jnp.float32),
        scratch_shapes=[
            pltpu.VMEM((1, BATCH), jnp.int32),
            pltpu.VMEM((NBUF, NUM_CLASSES, COLS), jnp.float32),
            pltpu.SemaphoreType.DMA,
            pltpu.SemaphoreType.DMA((NBUF,)),
        ],
    )(x)
    return oh_t.T


# speedup vs baseline: 7.7742x; 1.0751x over previous
import jax
import jax.numpy as jnp
from jax import lax
from jax.experimental import pallas as pl

NUM_CLASSES = 1000
BATCH = 16384
COLS = 1024


def _onehot_block(x_ref, out_ref):
    x = x_ref[...]
    rows = lax.broadcasted_iota(jnp.int32, (NUM_CLASSES, COLS), 0)
    out_ref[...] = jnp.where(x == rows, 1.0, 0.0).astype(jnp.float32)


def kernel(x):
    x = x.astype(jnp.int32).reshape(1, BATCH)
    grid = BATCH // COLS
    oh_t = pl.pallas_call(
        _onehot_block,
        grid=(grid,),
        in_specs=[pl.BlockSpec((1, COLS), lambda i: (0, i))],
        out_specs=pl.BlockSpec((NUM_CLASSES, COLS), lambda i: (0, i)),
        out_shape=jax.ShapeDtypeStruct((NUM_CLASSES, BATCH), jnp.float32),
    )(x)
    return oh_t.T
